# baseline (speedup 1.0000x reference)
import functools

import jax
import jax.numpy as jnp
import numpy as np
from jax import lax
from jax.experimental import pallas as pl
from jax.experimental.pallas import tpu as pltpu

NUM_BUCKETS = 32
MAX_DISTANCE = 128


def _bucket_ids(q_len, shape, iota_dim):
    t = lax.broadcasted_iota(jnp.int32, shape, iota_dim)
    n = (q_len - 1) - t
    num_buckets = NUM_BUCKETS // 2
    ret = jnp.where(n < 0, num_buckets, 0).astype(jnp.int32)
    n = jnp.abs(n)
    max_exact = num_buckets // 2
    is_small = n < max_exact
    val_if_large = max_exact + (
        jnp.log(n.astype(jnp.float32) / max_exact)
        / np.log(MAX_DISTANCE / max_exact)
        * (num_buckets - max_exact)
    ).astype(jnp.int32)
    val_if_large = jnp.minimum(val_if_large, num_buckets - 1)
    return ret + jnp.where(is_small, n, val_if_large)


def _tc_body(q_len, v_len, dim, bi, tw, embT_ref, out_ref, table_ref):
    pid = pl.program_id(0)

    @pl.when(pid == 0)
    def _build_table():
        bucket = _bucket_ids(q_len, (dim, tw), 1)
        acc = jnp.zeros((dim, tw), jnp.float32)
        for b in range(NUM_BUCKETS):
            acc = jnp.where(bucket == b, embT_ref[:, b : b + 1], acc)
        table_ref[...] = acc

    win = v_len + 128
    for r in range(bi):
        i = pid * bi + r
        s = (q_len - 1) - i
        k128 = (s // 128) * 128
        phi = s - k128
        w = table_ref[:, pl.ds(pl.multiple_of(k128, 128), win)]
        rolled = pltpu.roll(w, jnp.where(phi == 0, 0, win - phi), 1)
        out_ref[r] = rolled[:, :v_len]


def _table8_body(q_len, tw, dim, embT_ref, out_ref, full_ref):
    fw = full_ref.shape[1]
    bucket = _bucket_ids(q_len, (dim, fw), 1)
    acc = jnp.zeros((dim, fw), jnp.float32)
    for b in range(NUM_BUCKETS):
        acc = jnp.where(bucket == b, embT_ref[:, b : b + 1], acc)
    full_ref[...] = acc
    for rho in range(8):
        out_ref[rho] = full_ref[:, rho : rho + tw]


def _table8_call(embT, q_len, tw, dim):
    return pl.pallas_call(
        functools.partial(_table8_body, q_len, tw, dim),
        out_shape=jax.ShapeDtypeStruct((8, dim, tw), jnp.float32),
        scratch_shapes=[pltpu.VMEM((dim, tw + 128), jnp.float32)],
    )(embT)


def _sc_call(table8, q_len, v_len, dim):
    from jax.experimental.pallas import tpu_sc as plsc

    info = plsc.get_sparse_core_info()
    nw = info.num_cores * info.num_subcores
    nblk = nw // 8
    rows_per_w = q_len // nw
    ntr = dim // 8
    ntc = v_len // 128
    tw = table8.shape[2]

    @functools.partial(
        pl.kernel,
        out_type=jax.ShapeDtypeStruct((q_len, ntr, ntc, 8, 128), jnp.float32),
        mesh=plsc.VectorSubcoreMesh(core_axis_name="c", subcore_axis_name="s"),
        scratch_types=[
            pltpu.VMEM((dim, tw), jnp.float32),
            pltpu.SemaphoreType.DMA,
        ],
        compiler_params=pltpu.CompilerParams(use_tc_tiling_on_sc=False),
    )
    def body(tbl_hbm, out_hbm, t2, sem):
        wid = lax.axis_index("s") * info.num_cores + lax.axis_index("c")
        r0 = wid % 8
        blk = wid // 8
        rho = (q_len - 1 - r0) % 8
        pltpu.sync_copy(tbl_hbm.at[rho], t2)

        rg = 1

        def rowgrp(g, carry):
            handles = []
            for rr in range(rg):
                i = r0 + 8 * (blk * rows_per_w + g * rg + rr)
                s = (q_len - 1) - i
                cb = pl.multiple_of(s - rho, 8)
                for tr in range(ntr):
                    for tc in range(ntc):
                        handles.append(pltpu.async_copy(
                            t2.at[pl.ds(tr * 8, 8), pl.ds(cb + tc * 128, 128)],
                            out_hbm.at[i, tr, tc],
                            sem,
                        ))
            for h in handles:
                h.wait()
            return carry

        lax.fori_loop(0, rows_per_w // rg, rowgrp, 0)

    return body(table8)


def _tc_call(embT, q_len, v_len, dim, interpret=False):
    bi = 32
    tw = -(-(q_len + v_len) // 128) * 128
    return pl.pallas_call(
        functools.partial(_tc_body, q_len, v_len, dim, bi, tw),
        grid=(q_len // bi,),
        in_specs=[pl.BlockSpec((dim, NUM_BUCKETS), lambda i: (0, 0))],
        out_specs=pl.BlockSpec((bi, dim, v_len), lambda i: (i, 0, 0)),
        out_shape=jax.ShapeDtypeStruct((q_len, dim, v_len), jnp.float32),
        scratch_shapes=[pltpu.VMEM((dim, tw), jnp.float32)],
        interpret=interpret,
    )(embT)


def kernel(q, v, embeddings):
    q_len = q.shape[1]
    v_len = v.shape[1]
    dim = embeddings.shape[1]
    tw = -(-(q_len + v_len) // 128) * 128
    table8 = _table8_call(embeddings.T, q_len, tw, dim)
    out5 = _sc_call(table8, q_len, v_len, dim)
    res = out5.transpose(0, 1, 3, 2, 4).reshape(q_len, dim, v_len)
    return res.transpose(0, 2, 1)


# speedup vs baseline: 1.0226x; 1.0226x over previous
import functools

import jax
import jax.numpy as jnp
import numpy as np
from jax import lax
from jax.experimental import pallas as pl
from jax.experimental.pallas import tpu as pltpu

NUM_BUCKETS = 32
MAX_DISTANCE = 128


def _bucket_ids(q_len, shape, iota_dim):
    t = lax.broadcasted_iota(jnp.int32, shape, iota_dim)
    n = (q_len - 1) - t
    num_buckets = NUM_BUCKETS // 2
    ret = jnp.where(n < 0, num_buckets, 0).astype(jnp.int32)
    n = jnp.abs(n)
    max_exact = num_buckets // 2
    is_small = n < max_exact
    val_if_large = max_exact + (
        jnp.log(n.astype(jnp.float32) / max_exact)
        / np.log(MAX_DISTANCE / max_exact)
        * (num_buckets - max_exact)
    ).astype(jnp.int32)
    val_if_large = jnp.minimum(val_if_large, num_buckets - 1)
    return ret + jnp.where(is_small, n, val_if_large)


def _tc_body(q_len, v_len, dim, bi, tw, embT_ref, out_ref, table_ref):
    pid = pl.program_id(0)

    @pl.when(pid == 0)
    def _build_table():
        bucket = _bucket_ids(q_len, (dim, tw), 1)
        acc = jnp.zeros((dim, tw), jnp.float32)
        for b in range(NUM_BUCKETS):
            acc = jnp.where(bucket == b, embT_ref[:, b : b + 1], acc)
        table_ref[...] = acc

    win = v_len + 128
    for r in range(bi):
        i = pid * bi + r
        s = (q_len - 1) - i
        k128 = (s // 128) * 128
        phi = s - k128
        w = table_ref[:, pl.ds(pl.multiple_of(k128, 128), win)]
        rolled = pltpu.roll(w, jnp.where(phi == 0, 0, win - phi), 1)
        out_ref[r] = rolled[:, :v_len]


def _table8_body(q_len, tw, dim, embT_ref, out_ref, full_ref):
    fw = full_ref.shape[1]
    bucket = _bucket_ids(q_len, (dim, fw), 1)
    acc = jnp.zeros((dim, fw), jnp.float32)
    for b in range(NUM_BUCKETS):
        acc = jnp.where(bucket == b, embT_ref[:, b : b + 1], acc)
    full_ref[...] = acc
    for rho in range(8):
        out_ref[rho] = full_ref[:, rho : rho + tw]


def _table8_call(embT, q_len, tw, dim):
    return pl.pallas_call(
        functools.partial(_table8_body, q_len, tw, dim),
        out_shape=jax.ShapeDtypeStruct((8, dim, tw), jnp.float32),
        scratch_shapes=[pltpu.VMEM((dim, tw + 128), jnp.float32)],
    )(embT)


def _sc_call(table8, q_len, v_len, dim):
    from jax.experimental.pallas import tpu_sc as plsc

    info = plsc.get_sparse_core_info()
    nw = info.num_cores * info.num_subcores
    nblk = nw // 8
    rows_per_w = q_len // nw
    ntr = dim // 8
    ntc = v_len // 128
    tw = table8.shape[2]

    @functools.partial(
        pl.kernel,
        out_type=jax.ShapeDtypeStruct((q_len, ntr, ntc, 8, 128), jnp.float32),
        mesh=plsc.VectorSubcoreMesh(core_axis_name="c", subcore_axis_name="s"),
        scratch_types=[
            pltpu.VMEM((dim, tw), jnp.float32),
            pltpu.SemaphoreType.DMA,
        ],
        compiler_params=pltpu.CompilerParams(use_tc_tiling_on_sc=False),
    )
    def body(tbl_hbm, out_hbm, t2, sem):
        wid = lax.axis_index("s") * info.num_cores + lax.axis_index("c")
        r0 = wid % 8
        blk = wid // 8
        rho = (q_len - 1 - r0) % 8
        pltpu.sync_copy(tbl_hbm.at[rho], t2)

        rg = 2

        def rowgrp(g, carry):
            handles = []
            for rr in range(rg):
                i = r0 + 8 * (blk * rows_per_w + g * rg + rr)
                s = (q_len - 1) - i
                cb = pl.multiple_of(s - rho, 8)
                for tr in range(ntr):
                    for tc in range(ntc):
                        handles.append(pltpu.async_copy(
                            t2.at[pl.ds(tr * 8, 8), pl.ds(cb + tc * 128, 128)],
                            out_hbm.at[i, tr, tc],
                            sem,
                        ))
            for h in handles:
                h.wait()
            return carry

        lax.fori_loop(0, rows_per_w // rg, rowgrp, 0)

    return body(table8)


def _tc_call(embT, q_len, v_len, dim, interpret=False):
    bi = 32
    tw = -(-(q_len + v_len) // 128) * 128
    return pl.pallas_call(
        functools.partial(_tc_body, q_len, v_len, dim, bi, tw),
        grid=(q_len // bi,),
        in_specs=[pl.BlockSpec((dim, NUM_BUCKETS), lambda i: (0, 0))],
        out_specs=pl.BlockSpec((bi, dim, v_len), lambda i: (i, 0, 0)),
        out_shape=jax.ShapeDtypeStruct((q_len, dim, v_len), jnp.float32),
        scratch_shapes=[pltpu.VMEM((dim, tw), jnp.float32)],
        interpret=interpret,
    )(embT)


def kernel(q, v, embeddings):
    q_len = q.shape[1]
    v_len = v.shape[1]
    dim = embeddings.shape[1]
    tw = -(-(q_len + v_len) // 128) * 128
    table8 = _table8_call(embeddings.T, q_len, tw, dim)
    out5 = _sc_call(table8, q_len, v_len, dim)
    res = out5.transpose(0, 1, 3, 2, 4).reshape(q_len, dim, v_len)
    return res.transpose(0, 2, 1)


# speedup vs baseline: 1.0407x; 1.0178x over previous
import functools

import jax
import jax.numpy as jnp
import numpy as np
from jax import lax
from jax.experimental import pallas as pl
from jax.experimental.pallas import tpu as pltpu

NUM_BUCKETS = 32
MAX_DISTANCE = 128


def _bucket_ids(q_len, shape, iota_dim):
    t = lax.broadcasted_iota(jnp.int32, shape, iota_dim)
    n = (q_len - 1) - t
    num_buckets = NUM_BUCKETS // 2
    ret = jnp.where(n < 0, num_buckets, 0).astype(jnp.int32)
    n = jnp.abs(n)
    max_exact = num_buckets // 2
    is_small = n < max_exact
    val_if_large = max_exact + (
        jnp.log(n.astype(jnp.float32) / max_exact)
        / np.log(MAX_DISTANCE / max_exact)
        * (num_buckets - max_exact)
    ).astype(jnp.int32)
    val_if_large = jnp.minimum(val_if_large, num_buckets - 1)
    return ret + jnp.where(is_small, n, val_if_large)


def _tc_body(q_len, v_len, dim, bi, tw, embT_ref, out_ref, table_ref):
    pid = pl.program_id(0)

    @pl.when(pid == 0)
    def _build_table():
        bucket = _bucket_ids(q_len, (dim, tw), 1)
        acc = jnp.zeros((dim, tw), jnp.float32)
        for b in range(NUM_BUCKETS):
            acc = jnp.where(bucket == b, embT_ref[:, b : b + 1], acc)
        table_ref[...] = acc

    win = v_len + 128
    for r in range(bi):
        i = pid * bi + r
        s = (q_len - 1) - i
        k128 = (s // 128) * 128
        phi = s - k128
        w = table_ref[:, pl.ds(pl.multiple_of(k128, 128), win)]
        rolled = pltpu.roll(w, jnp.where(phi == 0, 0, win - phi), 1)
        out_ref[r] = rolled[:, :v_len]


def _table8_body(q_len, tw, dim, embT_ref, out_ref, full_ref):
    fw = full_ref.shape[1]
    bucket = _bucket_ids(q_len, (dim, fw), 1)
    acc = jnp.zeros((dim, fw), jnp.float32)
    for b in range(NUM_BUCKETS):
        acc = jnp.where(bucket == b, embT_ref[:, b : b + 1], acc)
    full_ref[...] = acc
    for rho in range(8):
        out_ref[rho] = full_ref[:, rho : rho + tw]


def _table8_call(embT, q_len, tw, dim):
    return pl.pallas_call(
        functools.partial(_table8_body, q_len, tw, dim),
        out_shape=jax.ShapeDtypeStruct((8, dim, tw), jnp.float32),
        scratch_shapes=[pltpu.VMEM((dim, tw + 128), jnp.float32)],
    )(embT)


def _sc_call(table8, q_len, v_len, dim):
    from jax.experimental.pallas import tpu_sc as plsc

    info = plsc.get_sparse_core_info()
    nw = info.num_cores * info.num_subcores
    nblk = nw // 8
    rows_per_w = q_len // nw
    ntr = dim // 8
    ntc = v_len // 128
    tw = table8.shape[2]

    span = v_len + 8 * rows_per_w
    span = -(-span // 8) * 8

    @functools.partial(
        pl.kernel,
        out_type=jax.ShapeDtypeStruct((q_len, ntr, ntc, 8, 128), jnp.float32),
        mesh=plsc.VectorSubcoreMesh(core_axis_name="c", subcore_axis_name="s"),
        scratch_types=[
            pltpu.VMEM((dim, span), jnp.float32),
            pltpu.SemaphoreType.DMA,
        ],
        compiler_params=pltpu.CompilerParams(use_tc_tiling_on_sc=False),
    )
    def body(tbl_hbm, out_hbm, t2, sem):
        wid = lax.axis_index("s") * info.num_cores + lax.axis_index("c")
        r0 = wid % 8
        blk = wid // 8
        rho = (q_len - 1 - r0) % 8
        i_last = r0 + 8 * (blk * rows_per_w + rows_per_w - 1)
        cb_min = pl.multiple_of(((q_len - 1) - i_last) - rho, 8)
        pltpu.sync_copy(tbl_hbm.at[rho, :, pl.ds(cb_min, span)], t2)

        rg = 2

        def rowgrp(g, carry):
            handles = []
            for rr in range(rg):
                i = r0 + 8 * (blk * rows_per_w + g * rg + rr)
                s = (q_len - 1) - i
                cb = pl.multiple_of(s - rho - cb_min, 8)
                for tr in range(ntr):
                    for tc in range(ntc):
                        handles.append(pltpu.async_copy(
                            t2.at[pl.ds(tr * 8, 8), pl.ds(cb + tc * 128, 128)],
                            out_hbm.at[i, tr, tc],
                            sem,
                        ))
            for h in handles:
                h.wait()
            return carry

        lax.fori_loop(0, rows_per_w // rg, rowgrp, 0)

    return body(table8)


def _tc_call(embT, q_len, v_len, dim, interpret=False):
    bi = 32
    tw = -(-(q_len + v_len) // 128) * 128
    return pl.pallas_call(
        functools.partial(_tc_body, q_len, v_len, dim, bi, tw),
        grid=(q_len // bi,),
        in_specs=[pl.BlockSpec((dim, NUM_BUCKETS), lambda i: (0, 0))],
        out_specs=pl.BlockSpec((bi, dim, v_len), lambda i: (i, 0, 0)),
        out_shape=jax.ShapeDtypeStruct((q_len, dim, v_len), jnp.float32),
        scratch_shapes=[pltpu.VMEM((dim, tw), jnp.float32)],
        interpret=interpret,
    )(embT)


def kernel(q, v, embeddings):
    q_len = q.shape[1]
    v_len = v.shape[1]
    dim = embeddings.shape[1]
    tw = -(-(q_len + v_len) // 128) * 128 + 128
    table8 = _table8_call(embeddings.T, q_len, tw, dim)
    out5 = _sc_call(table8, q_len, v_len, dim)
    res = out5.transpose(0, 1, 3, 2, 4).reshape(q_len, dim, v_len)
    return res.transpose(0, 2, 1)
